# baseline (speedup 1.0000x reference)
import jax
import jax.numpy as jnp
from jax import lax
from jax.experimental import pallas as pl
from jax.experimental.pallas import tpu as pltpu

_ROWS_BLK = 512
_G = 32
_L = 128
_T = 32
_STAGES = (1, 2, 4, 8, 16, 32, 64)
_QMIN = -128
_QMAX = 127
_INF = float("inf")


def _lane_cumsum(x, lanes):
    for s in _STAGES:
        x = x + jnp.where(lanes >= s, pltpu.roll(x, s, 1), 0)
    return x


def _take(x, idx):
    return jnp.take_along_axis(x, idx, axis=1, mode="promise_in_bounds")


def _compute_tables(g_ref, inv_ref, offs_ref, cnt_ref):
    k = g_ref[...]
    lanes = lax.broadcasted_iota(jnp.int32, (_T, _L), 1)
    zero = jnp.zeros((_T, _L), jnp.int32)

    rank = zero
    offs_row = zero
    cnt_row = zero
    offs = jnp.zeros((_T, 1), jnp.int32)
    for v in range(_G):
        eq = (k == v).astype(jnp.int32)
        pc_incl = _lane_cumsum(eq, lanes)
        cnt = pc_incl[:, _L - 1 : _L]
        rank = rank + jnp.where(eq == 1, offs + (pc_incl - eq), 0)
        is_v = lanes == v
        offs_row = offs_row + jnp.where(is_v, offs, 0)
        cnt_row = cnt_row + jnp.where(is_v, cnt, 0)
        offs = offs + cnt

    inv = zero
    for l in range(_L):
        src = jnp.sum(jnp.where(rank == l, lanes, 0), axis=1, keepdims=True)
        inv = inv + jnp.where(lanes == l, src, 0)

    inv_ref[...] = inv
    offs_ref[...] = offs_row
    cnt_ref[...] = cnt_row


def _observer_body(g_ref, x_ref, scale_ref, zp_ref, inv_ref, offs_ref,
                   cnt_ref):
    @pl.when(pl.program_id(0) == 0)
    def _():
        _compute_tables(g_ref, inv_ref, offs_ref, cnt_ref)

    r = x_ref.shape[0]
    lanes32 = lax.broadcasted_iota(jnp.int32, (_T, _L), 1)

    cnt_all = cnt_ref[...]
    offs_all = offs_ref[...]
    inv_all = inv_ref[...]
    maxc = jnp.max(cnt_all, axis=0, keepdims=True)
    mx32 = jnp.broadcast_to(maxc, (_T, _L))
    s_tot = jnp.sum(maxc, axis=1, keepdims=True)
    ncols1 = (s_tot + 95) // 96
    for n in range(8, 0, -1):
        fits = jnp.sum((maxc + n - 1) // n, axis=1, keepdims=True) <= _L
        ncols1 = jnp.where(fits, jnp.minimum(ncols1, n), ncols1)
    cap = (mx32 + jnp.broadcast_to(ncols1, (_T, _L)) - 1) // jnp.broadcast_to(
        ncols1, (_T, _L))
    gs = _lane_cumsum(cap, lanes32) - cap
    total = jnp.sum(cap[0:1, :], axis=1, keepdims=True)
    glabel = jnp.zeros((_T, _L), jnp.int32)
    for v in range(_G):
        glabel = glabel + (lanes32 >= gs[:, v : v + 1] + cap[:, v : v + 1])
    glabel = jnp.minimum(glabel, _G - 1)
    capl = _take(cap, glabel)
    gsl = _take(gs, glabel)
    ol_base = lanes32 - gsl
    in_canon = lanes32 < jnp.broadcast_to(total, (_T, _L))
    offs_g = _take(offs_all, glabel)
    cnt_g = _take(cnt_all, glabel)

    ncols_s = ncols1[0, 0]

    def bcast(row):
        return jnp.broadcast_to(row, (r, _L))

    def col_body(j, carry):
        acc = list(carry)
        o = j * capl + ol_base
        valid = (o < cnt_g) & in_canon
        q = jnp.clip(offs_g + o, 0, _L - 1)
        idx_all = _take(inv_all, q)
        am_all = jnp.where(valid, 0.0, _INF)
        for t in range(_T):
            xt = x_ref[:, t * _L : (t + 1) * _L]
            xs = _take(xt, bcast(idx_all[t : t + 1, :]))
            am = bcast(am_all[t : t + 1, :])
            c = t % 2
            acc[2 * c] = jnp.minimum(acc[2 * c], xs + am)
            acc[2 * c + 1] = jnp.maximum(acc[2 * c + 1], xs - am)
        return tuple(acc)

    acc0 = (
        jnp.full((r, _L), _INF, jnp.float32),
        jnp.full((r, _L), -_INF, jnp.float32),
        jnp.full((r, _L), _INF, jnp.float32),
        jnp.full((r, _L), -_INF, jnp.float32),
    )
    amin0, amax0, amin1, amax1 = lax.fori_loop(0, ncols_s, col_body, acc0)
    accmin = jnp.minimum(amin0, amin1)
    accmax = jnp.maximum(amax0, amax1)

    for s in _STAGES:
        lab_sh = pltpu.roll(glabel, s, 1)
        pvalid = (lanes32 >= s) & (lab_sh == glabel)
        ps = jnp.where(pvalid, lanes32 - s, lanes32)
        psr = bcast(ps[0:1, :])
        accmin = jnp.minimum(accmin, _take(accmin, psr))
        accmax = jnp.maximum(accmax, _take(accmax, psr))
    exi = jnp.clip(gs + cap - 1, 0, _L - 1)
    em = jnp.where((lanes32 < _G) & (mx32 > 0), 0.0, _INF)
    exr = bcast(exi[0:1, :])
    emr = bcast(em[0:1, :])
    gmin = (_take(accmin, exr) + emr)[:, :_G]
    gmax = (_take(accmax, exr) - emr)[:, :_G]

    min_v = jnp.minimum(gmin, 0.0)
    max_v = jnp.maximum(gmax, 0.0)
    scale = (max_v - min_v) / float(_QMAX - _QMIN)
    scale = jnp.maximum(scale, jnp.finfo(jnp.float32).eps)
    zp = jnp.clip(jnp.round(_QMIN - min_v / scale), _QMIN, _QMAX).astype(jnp.int32)
    scale_ref[...] = scale
    zp_ref[...] = zp


@jax.jit
def kernel(observed, g_idx):
    rows, cols = observed.shape
    g2d = g_idx.reshape(_T, _L)
    grid = (rows // _ROWS_BLK,)
    out_shapes = (
        jax.ShapeDtypeStruct((rows, _G), jnp.float32),
        jax.ShapeDtypeStruct((rows, _G), jnp.int32),
    )
    scale, zp = pl.pallas_call(
        _observer_body,
        grid=grid,
        in_specs=[
            pl.BlockSpec((_T, _L), lambda i: (0, 0)),
            pl.BlockSpec((_ROWS_BLK, cols), lambda i: (i, 0)),
        ],
        out_specs=(
            pl.BlockSpec((_ROWS_BLK, _G), lambda i: (i, 0)),
            pl.BlockSpec((_ROWS_BLK, _G), lambda i: (i, 0)),
        ),
        out_shape=out_shapes,
        scratch_shapes=[
            pltpu.VMEM((_T, _L), jnp.int32),
            pltpu.VMEM((_T, _L), jnp.int32),
            pltpu.VMEM((_T, _L), jnp.int32),
        ],
        compiler_params=pltpu.CompilerParams(
            dimension_semantics=("arbitrary",),
        ),
    )(g2d, observed)
    return scale, zp


# speedup vs baseline: 1.0995x; 1.0995x over previous
import jax
import jax.numpy as jnp
from jax import lax
from jax.experimental import pallas as pl
from jax.experimental.pallas import tpu as pltpu

_ROWS_BLK = 1024
_G = 32
_L = 128
_T = 32
_STAGES = (1, 2, 4, 8, 16, 32, 64)
_QMIN = -128
_QMAX = 127
_INF = float("inf")


def _lane_cumsum(x, lanes):
    for s in _STAGES:
        x = x + jnp.where(lanes >= s, pltpu.roll(x, s, 1), 0)
    return x


def _take(x, idx):
    return jnp.take_along_axis(x, idx, axis=1, mode="promise_in_bounds")


def _compute_tables(g_ref, inv_ref, offs_ref, cnt_ref):
    k = g_ref[...]
    lanes = lax.broadcasted_iota(jnp.int32, (_T, _L), 1)
    zero = jnp.zeros((_T, _L), jnp.int32)

    rank = zero
    offs_row = zero
    cnt_row = zero
    offs = jnp.zeros((_T, 1), jnp.int32)
    for v in range(_G):
        eq = (k == v).astype(jnp.int32)
        pc_incl = _lane_cumsum(eq, lanes)
        cnt = pc_incl[:, _L - 1 : _L]
        rank = rank + jnp.where(eq == 1, offs + (pc_incl - eq), 0)
        is_v = lanes == v
        offs_row = offs_row + jnp.where(is_v, offs, 0)
        cnt_row = cnt_row + jnp.where(is_v, cnt, 0)
        offs = offs + cnt

    inv = zero
    for l in range(_L):
        src = jnp.sum(jnp.where(rank == l, lanes, 0), axis=1, keepdims=True)
        inv = inv + jnp.where(lanes == l, src, 0)

    inv_ref[...] = inv
    offs_ref[...] = offs_row
    cnt_ref[...] = cnt_row


def _observer_body(g_ref, x_ref, scale_ref, zp_ref, inv_ref, offs_ref,
                   cnt_ref):
    @pl.when(pl.program_id(0) == 0)
    def _():
        _compute_tables(g_ref, inv_ref, offs_ref, cnt_ref)

    r = x_ref.shape[0]
    lanes32 = lax.broadcasted_iota(jnp.int32, (_T, _L), 1)

    cnt_all = cnt_ref[...]
    offs_all = offs_ref[...]
    inv_all = inv_ref[...]
    maxc = jnp.max(cnt_all, axis=0, keepdims=True)
    mx32 = jnp.broadcast_to(maxc, (_T, _L))
    s_tot = jnp.sum(maxc, axis=1, keepdims=True)
    ncols1 = (s_tot + 95) // 96
    for n in range(8, 0, -1):
        fits = jnp.sum((maxc + n - 1) // n, axis=1, keepdims=True) <= _L
        ncols1 = jnp.where(fits, jnp.minimum(ncols1, n), ncols1)
    cap = (mx32 + jnp.broadcast_to(ncols1, (_T, _L)) - 1) // jnp.broadcast_to(
        ncols1, (_T, _L))
    gs = _lane_cumsum(cap, lanes32) - cap
    total = jnp.sum(cap[0:1, :], axis=1, keepdims=True)
    glabel = jnp.zeros((_T, _L), jnp.int32)
    for v in range(_G):
        glabel = glabel + (lanes32 >= gs[:, v : v + 1] + cap[:, v : v + 1])
    glabel = jnp.minimum(glabel, _G - 1)
    capl = _take(cap, glabel)
    gsl = _take(gs, glabel)
    ol_base = lanes32 - gsl
    in_canon = lanes32 < jnp.broadcast_to(total, (_T, _L))
    offs_g = _take(offs_all, glabel)
    cnt_g = _take(cnt_all, glabel)

    ncols_s = ncols1[0, 0]

    def bcast(row):
        return jnp.broadcast_to(row, (r, _L))

    def col_body(j, carry):
        accmin, accmax = carry
        o = j * capl + ol_base
        valid = (o < cnt_g) & in_canon
        q = jnp.clip(offs_g + o, 0, _L - 1)
        idx_all = _take(inv_all, q)
        am_all = jnp.where(valid, 0.0, _INF)
        for t in range(_T):
            xt = x_ref[:, t * _L : (t + 1) * _L]
            xs = _take(xt, bcast(idx_all[t : t + 1, :]))
            am = bcast(am_all[t : t + 1, :])
            accmin = jnp.minimum(accmin, xs + am)
            accmax = jnp.maximum(accmax, xs - am)
        return accmin, accmax

    acc0 = (
        jnp.full((r, _L), _INF, jnp.float32),
        jnp.full((r, _L), -_INF, jnp.float32),
    )
    accmin, accmax = lax.fori_loop(0, ncols_s, col_body, acc0)

    for s in _STAGES:
        lab_sh = pltpu.roll(glabel, s, 1)
        pvalid = (lanes32 >= s) & (lab_sh == glabel)
        ps = jnp.where(pvalid, lanes32 - s, lanes32)
        psr = bcast(ps[0:1, :])
        accmin = jnp.minimum(accmin, _take(accmin, psr))
        accmax = jnp.maximum(accmax, _take(accmax, psr))
    exi = jnp.clip(gs + cap - 1, 0, _L - 1)
    em = jnp.where((lanes32 < _G) & (mx32 > 0), 0.0, _INF)
    exr = bcast(exi[0:1, :])
    emr = bcast(em[0:1, :])
    gmin = (_take(accmin, exr) + emr)[:, :_G]
    gmax = (_take(accmax, exr) - emr)[:, :_G]

    min_v = jnp.minimum(gmin, 0.0)
    max_v = jnp.maximum(gmax, 0.0)
    scale = (max_v - min_v) / float(_QMAX - _QMIN)
    scale = jnp.maximum(scale, jnp.finfo(jnp.float32).eps)
    zp = jnp.clip(jnp.round(_QMIN - min_v / scale), _QMIN, _QMAX).astype(jnp.int32)
    scale_ref[...] = scale
    zp_ref[...] = zp


@jax.jit
def kernel(observed, g_idx):
    rows, cols = observed.shape
    g2d = g_idx.reshape(_T, _L)
    grid = (rows // _ROWS_BLK,)
    out_shapes = (
        jax.ShapeDtypeStruct((rows, _G), jnp.float32),
        jax.ShapeDtypeStruct((rows, _G), jnp.int32),
    )
    scale, zp = pl.pallas_call(
        _observer_body,
        grid=grid,
        in_specs=[
            pl.BlockSpec((_T, _L), lambda i: (0, 0)),
            pl.BlockSpec((_ROWS_BLK, cols), lambda i: (i, 0)),
        ],
        out_specs=(
            pl.BlockSpec((_ROWS_BLK, _G), lambda i: (i, 0)),
            pl.BlockSpec((_ROWS_BLK, _G), lambda i: (i, 0)),
        ),
        out_shape=out_shapes,
        scratch_shapes=[
            pltpu.VMEM((_T, _L), jnp.int32),
            pltpu.VMEM((_T, _L), jnp.int32),
            pltpu.VMEM((_T, _L), jnp.int32),
        ],
        compiler_params=pltpu.CompilerParams(
            dimension_semantics=("arbitrary",),
        ),
    )(g2d, observed)
    return scale, zp
